# Initial kernel scaffold; baseline (speedup 1.0000x reference)
import jax
import jax.numpy as jnp
from jax.experimental import pallas as pl


def kernel(F, edge_index, Wg, bg, Wfc, bfc):
    raise NotImplementedError("write your pallas kernel here")


# speedup vs baseline: 11.0242x; 11.0242x over previous
import functools

import jax
import jax.numpy as jnp
from jax import lax
from jax.experimental import pallas as pl
from jax.experimental.pallas import tpu as pltpu
from jax.experimental.pallas import tpu_sc as plsc

N_NODES = 10000
D = 128
DH = D // 2
OUT_DIM = 16
N_PAD = 10240
E = 320000
NC = 2
NS = 16
NW = NC * NS
Q = N_PAD
E_PAD = NW * Q
CHUNK = 128
NCHUNK = Q // CHUNK
ROWS_PER_TILE = N_PAD // NS

_sc_mesh = plsc.VectorSubcoreMesh(core_axis_name="c", subcore_axis_name="s")


@functools.partial(
    pl.kernel,
    mesh=_sc_mesh,
    out_type=[
        jax.ShapeDtypeStruct((NC, N_PAD, 1), jnp.float32),
        jax.ShapeDtypeStruct((NC, N_PAD, 1), jnp.float32),
    ],
    scratch_types=[
        pltpu.VMEM((NCHUNK, CHUNK), jnp.int32),
        pltpu.VMEM((NCHUNK, CHUNK), jnp.int32),
        pltpu.VMEM((CHUNK, 1), jnp.float32),
        pltpu.VMEM_SHARED((N_PAD, 1), jnp.float32),
        pltpu.VMEM_SHARED((N_PAD, 1), jnp.float32),
    ],
)
def _deg_kernel(src_hbm, dst_hbm, ones_hbm, outs_hbm, outd_hbm,
                src_v, dst_v, ones_v, hs_sh, hd_sh):
    c = lax.axis_index("c")
    s = lax.axis_index("s")
    w = s * NC + c

    pltpu.sync_copy(src_hbm.at[pl.ds(w * NCHUNK, NCHUNK)], src_v)
    pltpu.sync_copy(dst_hbm.at[pl.ds(w * NCHUNK, NCHUNK)], dst_v)
    pltpu.sync_copy(ones_hbm.at[pl.ds(0, CHUNK)], ones_v)

    for k in range(ROWS_PER_TILE // CHUNK):
        dst_slice = pl.ds(s * ROWS_PER_TILE + k * CHUNK, CHUNK)
        pltpu.sync_copy(ones_hbm.at[pl.ds(CHUNK, CHUNK)], hs_sh.at[dst_slice])
        pltpu.sync_copy(ones_hbm.at[pl.ds(CHUNK, CHUNK)], hd_sh.at[dst_slice])
    plsc.subcore_barrier()

    def _count(j, _):
        pltpu.sync_copy(ones_v, hs_sh.at[src_v.at[j]], add=True)
        pltpu.sync_copy(ones_v, hd_sh.at[dst_v.at[j]], add=True)
        return _
    lax.fori_loop(0, NCHUNK, _count, None)
    plsc.subcore_barrier()

    sl = pl.ds(s * ROWS_PER_TILE, ROWS_PER_TILE)
    pltpu.sync_copy(hs_sh.at[sl], outs_hbm.at[c, sl])
    pltpu.sync_copy(hd_sh.at[sl], outd_hbm.at[c, sl])


NSTAGE = 2
SCHUNK = NCHUNK // NSTAGE


@functools.partial(
    pl.kernel,
    mesh=_sc_mesh,
    out_type=jax.ShapeDtypeStruct((NC, N_PAD, D), jnp.float32),
    scratch_types=[
        pltpu.VMEM((SCHUNK, CHUNK), jnp.int32),
        pltpu.VMEM((SCHUNK, CHUNK), jnp.int32),
        pltpu.VMEM((CHUNK, D), jnp.float32),
        pltpu.VMEM((CHUNK, D), jnp.float32),
        pltpu.VMEM_SHARED((N_PAD, D), jnp.float32),
        pltpu.SemaphoreType.DMA,
        pltpu.SemaphoreType.DMA,
    ],
)
def _scatter_kernel(t_hbm, src_hbm, dst_hbm, zero_hbm, out_hbm,
                    src_v, dst_v, rows0, rows1, agg_sh, gsem0, gsem1):
    c = lax.axis_index("c")
    s = lax.axis_index("s")
    w = s * NC + c
    rows = (rows0, rows1)
    gsems = (gsem0, gsem1)

    for k in range(ROWS_PER_TILE // CHUNK):
        pltpu.sync_copy(
            zero_hbm, agg_sh.at[pl.ds(s * ROWS_PER_TILE + k * CHUNK, CHUNK)])
    plsc.subcore_barrier()

    for st in range(NSTAGE):
        esl = pl.ds(w * NCHUNK + st * SCHUNK, SCHUNK)
        pltpu.sync_copy(src_hbm.at[esl], src_v)
        pltpu.sync_copy(dst_hbm.at[esl], dst_v)

        for b in range(2):
            pltpu.make_async_copy(t_hbm.at[src_v.at[b]], rows[b], gsems[b]).start()

        def _step(g, _):
            for b in range(2):
                j = 2 * g + b
                pltpu.make_async_copy(
                    t_hbm.at[src_v.at[j]], rows[b], gsems[b]).wait()
                pltpu.sync_copy(rows[b], agg_sh.at[dst_v.at[j]], add=True)

                @pl.when(j + 2 < SCHUNK)
                def _():
                    pltpu.make_async_copy(
                        t_hbm.at[src_v.at[j + 2]], rows[b], gsems[b]).start()
            return _
        lax.fori_loop(0, SCHUNK // 2, _step, None)

    plsc.subcore_barrier()
    sl = pl.ds(s * ROWS_PER_TILE, ROWS_PER_TILE)
    pltpu.sync_copy(agg_sh.at[sl], out_hbm.at[c, sl])


def _csrc(p0, p1):
    return lax.rsqrt(jnp.maximum(p0 + p1 + 1.0, 1.0))


BN = 1024


def _mm_first_body(h_ref, p0_ref, p1_ref, w_ref, o_ref):
    cs = _csrc(p0_ref[...], p1_ref[...])
    o_ref[...] = jnp.dot(h_ref[...] * cs, w_ref[...],
                         preferred_element_type=jnp.float32)


def _mm_first(h, p0, p1, w):
    return pl.pallas_call(
        _mm_first_body,
        grid=(N_PAD // BN,),
        in_specs=[
            pl.BlockSpec((BN, D), lambda i: (i, 0)),
            pl.BlockSpec((BN, 1), lambda i: (i, 0)),
            pl.BlockSpec((BN, 1), lambda i: (i, 0)),
            pl.BlockSpec((D, D), lambda i: (0, 0)),
        ],
        out_specs=pl.BlockSpec((BN, D), lambda i: (i, 0)),
        out_shape=jax.ShapeDtypeStruct((N_PAD, D), jnp.float32),
    )(h, p0, p1, w)


def _mm_layer_body(a0_ref, a1_ref, t_ref,
                   pd0_ref, pd1_ref, ps0_ref, ps1_ref, b_ref, w_ref, o_ref):
    cd = _csrc(pd0_ref[...], pd1_ref[...])
    agg = a0_ref[...] + a1_ref[...] + t_ref[...]
    h = jnp.maximum(agg * cd + b_ref[...], 0.0)
    cs = _csrc(ps0_ref[...], ps1_ref[...])
    o_ref[...] = jnp.dot(h * cs, w_ref[...], preferred_element_type=jnp.float32)


def _mm_layer(a0, a1, t, pd0, pd1, ps0, ps1, b, w):
    full = pl.BlockSpec((BN, D), lambda i: (i, 0))
    col = pl.BlockSpec((BN, 1), lambda i: (i, 0))
    return pl.pallas_call(
        _mm_layer_body,
        grid=(N_PAD // BN,),
        in_specs=[
            full, full, full,
            col, col, col, col,
            pl.BlockSpec((1, D), lambda i: (0, 0)),
            pl.BlockSpec((D, D), lambda i: (0, 0)),
        ],
        out_specs=pl.BlockSpec((BN, D), lambda i: (i, 0)),
        out_shape=jax.ShapeDtypeStruct((N_PAD, D), jnp.float32),
    )(a0, a1, t, pd0, pd1, ps0, ps1, b, w)


BF = 400


def _final_body(a0_ref, a1_ref, t_ref,
                pd0_ref, pd1_ref, b_ref, wfc_ref, o_ref, acc_ref):
    i = pl.program_id(0)

    @pl.when(i == 0)
    def _():
        acc_ref[...] = jnp.zeros_like(acc_ref)

    cd = _csrc(pd0_ref[...], pd1_ref[...])
    agg = a0_ref[...] + a1_ref[...] + t_ref[...]
    h = jnp.maximum(agg * cd + b_ref[...], 0.0)
    for k in range(OUT_DIM):
        acc_ref[k:k + 1, :] += jnp.sum(wfc_ref[k] * h, axis=0, keepdims=True)

    @pl.when(i == N_NODES // BF - 1)
    def _():
        o_ref[...] = jnp.sum(acc_ref[...], axis=1, keepdims=True)


def _final(a0, a1, t, pd0, pd1, b, wfc3):
    full = pl.BlockSpec((BF, D), lambda i: (i, 0))
    col = pl.BlockSpec((BF, 1), lambda i: (i, 0))
    return pl.pallas_call(
        _final_body,
        grid=(N_NODES // BF,),
        in_specs=[
            full, full, full,
            col, col,
            pl.BlockSpec((1, D), lambda i: (0, 0)),
            pl.BlockSpec((OUT_DIM, BF, D), lambda i: (0, i, 0)),
        ],
        out_specs=pl.BlockSpec((OUT_DIM, 1), lambda i: (0, 0)),
        out_shape=jax.ShapeDtypeStruct((OUT_DIM, 1), jnp.float32),
        scratch_shapes=[pltpu.VMEM((OUT_DIM, D), jnp.float32)],
    )(a0, a1, t, pd0, pd1, b, wfc3)


def kernel(F, edge_index, Wg, bg, Wfc, bfc):
    src = edge_index[0].astype(jnp.int32)
    dst = edge_index[1].astype(jnp.int32)
    npad_e = E_PAD - E
    fill = (N_NODES + (jnp.arange(npad_e, dtype=jnp.int32) % (N_PAD - N_NODES)))
    src_p = jnp.concatenate([src, fill])
    dst_p = jnp.concatenate([dst, fill])
    src2d = src_p.reshape(E_PAD // CHUNK, CHUNK)
    dst2d = dst_p.reshape(E_PAD // CHUNK, CHUNK)

    F_pad = jnp.pad(F, ((0, N_PAD - N_NODES), (0, 0)))

    ones_zeros = jnp.concatenate(
        [jnp.ones((CHUNK, 1), jnp.float32), jnp.zeros((CHUNK, 1), jnp.float32)])
    zero_rows = jnp.zeros((CHUNK, D), jnp.float32)

    dsp, ddp = _deg_kernel(src2d, dst2d, ones_zeros)
    ps0, ps1 = dsp[0], dsp[1]
    pd0, pd1 = ddp[0], ddp[1]

    t = _mm_first(F_pad, ps0, ps1, Wg[0])
    for i in range(2):
        aggs = _scatter_kernel(t, src2d, dst2d, zero_rows)
        t = _mm_layer(aggs[0], aggs[1], t,
                      pd0, pd1, ps0, ps1, bg[i].reshape(1, D), Wg[i + 1])
    aggs = _scatter_kernel(t, src2d, dst2d, zero_rows)
    res = _final(aggs[0], aggs[1], t,
                 pd0, pd1, bg[2].reshape(1, D),
                 Wfc.reshape(OUT_DIM, N_NODES, D))
    return res[:, 0] + bfc
